# baseline (speedup 1.0000x reference)
import functools

import jax
import jax.numpy as jnp
from jax import lax
from jax.experimental import pallas as pl
from jax.experimental.pallas import tpu as pltpu
from jax.experimental.pallas import tpu_sc as plsc

_SIGMA = 0.5
_L = 16


def _y_matmul_body(x_ref, w_ref, o_ref):
    o_ref[0] = jnp.dot(
        x_ref[...], w_ref[0], preferred_element_type=jnp.float32
    ).astype(jnp.bfloat16)


def _rsqrt(d2):
    i = plsc.bitcast(d2, jnp.int32)
    i = jnp.int32(0x5F3759DF) - (i >> 1)
    y = plsc.bitcast(i, jnp.float32)
    for _ in range(2):
        y = y * (1.5 - 0.5 * d2 * y * y)
    return y


@functools.lru_cache(maxsize=None)
def _make_sc_kernel(N, E, K, D):
    NC, NS = 2, 16
    NW = NC * NS
    EPW = E // NW
    GPW = EPW // _L
    RPT = N // NS
    ZR = 25
    R8 = D // _L
    EB = 2000
    GPB = EB // _L
    CAP = 256
    NSLOT = 4

    mesh = plsc.VectorSubcoreMesh(core_axis_name="c", subcore_axis_name="s")

    @functools.partial(
        pl.kernel,
        out_type=jax.ShapeDtypeStruct((NC, N, D), jnp.float32),
        mesh=mesh,
        compiler_params=pltpu.CompilerParams(
            use_tc_tiling_on_sc=False, needs_layout_passes=False),
        scratch_types=[
            pltpu.VMEM((3 * _L,), jnp.float32),
            pltpu.VMEM((EB,), jnp.int32),
            pltpu.VMEM((EB,), jnp.int32),
            pltpu.VMEM((2, 6, _L), jnp.float32),
            pltpu.VMEM((2, CAP), jnp.int32),
            pltpu.VMEM((2, CAP), jnp.float32),
            pltpu.VMEM((2, CAP), jnp.int32),
            pltpu.VMEM((NSLOT, _L, D), jnp.bfloat16),
            pltpu.VMEM((NSLOT, _L, D), jnp.float32),
            pltpu.VMEM((ZR, D), jnp.float32),
            pltpu.VMEM_SHARED((N, D), jnp.float32),
            pltpu.SemaphoreType.DMA,
            pltpu.SemaphoreType.DMA,
            pltpu.SemaphoreType.DMA,
            pltpu.SemaphoreType.DMA,
            pltpu.SemaphoreType.DMA,
            pltpu.SemaphoreType.DMA,
            pltpu.SemaphoreType.DMA,
            pltpu.SemaphoreType.DMA,
            pltpu.SemaphoreType.DMA,
        ],
    )
    def sc(y_hbm, src_hbm, dst_hbm, px_hbm, py_hbm, pz_hbm, kp_hbm, out_hbm,
           kpv, srcv, dstv, posb, idxb, inflb, dstb, rowb, scatb, zerov,
           outacc, psem, g0, g1, g2, g3, s0, s1, s2, s3):
        gsems = [g0, g1, g2, g3]
        ssems = [s0, s1, s2, s3]
        cid = lax.axis_index("c")
        sid = lax.axis_index("s")
        wid = cid * NS + sid
        ebase = wid * EPW

        pltpu.sync_copy(kp_hbm, kpv)

        def zb(i, _):
            for r in range(R8):
                zerov[i, pl.ds(r * _L, _L)] = jnp.zeros((_L,), jnp.float32)
            return 0
        lax.fori_loop(0, ZR, zb, 0)
        base = sid * RPT

        def zc(i, _):
            pltpu.sync_copy(zerov, outacc.at[pl.ds(base + i * ZR, ZR)])
            return 0
        lax.fori_loop(0, RPT // ZR, zc, 0)
        plsc.subcore_barrier()

        kxv = kpv[pl.ds(0, _L)]
        kyv = kpv[pl.ds(_L, _L)]
        kzv = kpv[pl.ds(2 * _L, _L)]
        kps = [(kxv[k], kyv[k], kzv[k]) for k in range(K)]

        def issue_pos(gl, b):
            sv = srcv[pl.ds(gl * _L, _L)]
            dv = dstv[pl.ds(gl * _L, _L)]
            pltpu.async_copy(px_hbm.at[sv], posb.at[b].at[0], psem)
            pltpu.async_copy(py_hbm.at[sv], posb.at[b].at[1], psem)
            pltpu.async_copy(pz_hbm.at[sv], posb.at[b].at[2], psem)
            pltpu.async_copy(px_hbm.at[dv], posb.at[b].at[3], psem)
            pltpu.async_copy(py_hbm.at[dv], posb.at[b].at[4], psem)
            pltpu.async_copy(pz_hbm.at[dv], posb.at[b].at[5], psem)

        def fire(pbuf, i, tb):
            iv = idxb[pbuf, pl.ds(i * _L, _L)]
            slot = (tb + i) % NSLOT
            for s in range(NSLOT):
                @pl.when(slot == s)
                def _():
                    pltpu.async_copy(y_hbm.at[iv], rowb.at[s], gsems[s])

        def process(i, tb, inflv, dvec):
            slot = (tb + i) % NSLOT
            for s in range(NSLOT):
                @pl.when(slot == s)
                def _():
                    @pl.when(tb + i >= NSLOT)
                    def _():
                        pltpu.make_async_copy(
                            out_hbm.at[0].at[pl.ds(0, _L)],
                            scatb.at[s], ssems[s]).wait()
                    pltpu.make_async_copy(
                        y_hbm.at[pl.ds(0, _L)], rowb.at[s], gsems[s]).wait()
                    for j in range(_L):
                        f = inflv[j]
                        for r in range(D // 32):
                            v = rowb[s, j, pl.ds(r * 32, 32)]
                            a, bq = plsc.unpack(
                                v, format=plsc.PackFormat.INTERLEAVED)
                            scatb[s, j, pl.ds(r * 32, _L)] = (
                                a.astype(jnp.float32) * f)
                            scatb[s, j, pl.ds(r * 32 + _L, _L)] = (
                                bq.astype(jnp.float32) * f)
                    pltpu.async_copy(scatb.at[s], outacc.at[dvec],
                                     ssems[s], add=True)

        def run_batches(pbuf, pnb, tb):
            def bloop(i, _):
                @pl.when(i + (NSLOT - 1) < pnb)
                def _():
                    fire(pbuf, i + (NSLOT - 1), tb)
                inflv = inflb[pbuf, pl.ds(i * _L, _L)]
                dvec = dstb[pbuf, pl.ds(i * _L, _L)]
                process(i, tb, inflv, dvec)
                return 0
            lax.fori_loop(0, pnb, bloop, 0)

        def body(g, carry):
            rem, pnb, tb = carry
            gl = g % GPB
            b = g % 2
            pb = 1 - b

            run_batches(pb, pnb, tb)

            @pl.when(gl == 0)
            def _():
                blk = g // GPB
                pltpu.sync_copy(src_hbm.at[pl.ds(ebase + blk * EB, EB)], srcv)
                pltpu.sync_copy(dst_hbm.at[pl.ds(ebase + blk * EB, EB)], dstv)
                issue_pos(0, b)

            sv = srcv[pl.ds(gl * _L, _L)]
            dv = dstv[pl.ds(gl * _L, _L)]

            for c in range(6):
                pltpu.make_async_copy(
                    px_hbm.at[pl.ds(0, _L)], posb.at[b].at[c], psem).wait()
            relx = posb[b, 0, :] - posb[b, 3, :]
            rely = posb[b, 1, :] - posb[b, 4, :]
            relz = posb[b, 2, :] - posb[b, 5, :]

            @pl.when(jnp.logical_and(g + 1 < GPW, gl + 1 < GPB))
            def _():
                issue_pos(gl + 1, 1 - b)

            cnt = rem
            off = pnb * _L
            idxb[b, pl.ds(0, _L)] = idxb[pb, pl.ds(off, _L)]
            inflb[b, pl.ds(0, _L)] = inflb[pb, pl.ds(off, _L)]
            dstb[b, pl.ds(0, _L)] = dstb[pb, pl.ds(off, _L)]

            for k in range(K):
                kx, ky, kz = kps[k]
                dx = relx - kx
                dy = rely - ky
                dz = relz - kz
                d2 = dx * dx + dy * dy + dz * dz + 1e-12
                dist = d2 * _rsqrt(d2)
                infl = jnp.maximum(0.0, 1.0 - dist * (1.0 / _SIGMA))
                m = infl > 0.0
                plsc.store_compressed(
                    idxb.at[b].at[pl.ds(cnt, _L)], sv + k * N, mask=m)
                plsc.store_compressed(
                    inflb.at[b].at[pl.ds(cnt, _L)], infl, mask=m)
                plsc.store_compressed(
                    dstb.at[b].at[pl.ds(cnt, _L)], dv, mask=m)
                cnt = cnt + plsc.all_reduce_population_count(m)[0]

            nbat = cnt // _L
            tbn = tb + pnb

            def prime(i, _):
                fire(b, i, tbn)
                return 0
            lax.fori_loop(0, jnp.minimum(nbat, NSLOT - 1), prime, 0)

            return cnt - nbat * _L, nbat, tbn

        rem, pnb, tb = lax.fori_loop(
            0, GPW, body, (jnp.int32(0), jnp.int32(0), jnp.int32(0)))

        lb = 1 - (GPW % 2)
        run_batches(lb, pnb, tb)

        @pl.when(rem > 0)
        def _():
            off = pnb * _L
            lane = lax.iota(jnp.int32, _L)
            m = lane < rem
            iv = jnp.where(m, idxb[lb, pl.ds(off, _L)], 0)
            fv = jnp.where(m, inflb[lb, pl.ds(off, _L)], 0.0)
            dvv = jnp.where(m, dstb[lb, pl.ds(off, _L)], 0)
            idxb[lb, pl.ds(off, _L)] = iv
            inflb[lb, pl.ds(off, _L)] = fv
            dstb[lb, pl.ds(off, _L)] = dvv
            fire(lb, pnb, tb)
            process(pnb, tb, fv, dvv)

        tbf = tb + pnb + jnp.where(rem > 0, 1, 0)
        for s in range(NSLOT):
            @pl.when(tbf > s)
            def _():
                pltpu.make_async_copy(
                    out_hbm.at[0].at[pl.ds(0, _L)], scatb.at[s], ssems[s]).wait()
        plsc.subcore_barrier()

        pltpu.sync_copy(outacc.at[pl.ds(base, RPT)],
                        out_hbm.at[cid].at[pl.ds(base, RPT)])

    return sc


def kernel(x, pos, edge_index, kernel_points, weights):
    N, D_IN = x.shape
    K = kernel_points.shape[0]
    D_OUT = weights.shape[2]
    E = edge_index.shape[1]

    w_perm = weights.reshape(
        K, D_IN, D_OUT // 32, 2, _L).swapaxes(3, 4).reshape(K, D_IN, D_OUT)
    y = pl.pallas_call(
        _y_matmul_body,
        grid=(K,),
        in_specs=[
            pl.BlockSpec((N, D_IN), lambda k: (0, 0)),
            pl.BlockSpec((1, D_IN, D_OUT), lambda k: (k, 0, 0)),
        ],
        out_specs=pl.BlockSpec((1, N, D_OUT), lambda k: (k, 0, 0)),
        out_shape=jax.ShapeDtypeStruct((K, N, D_OUT), jnp.bfloat16),
    )(x, w_perm)

    src = edge_index[0]
    dst = edge_index[1]
    kp = jnp.zeros((3, _L), jnp.float32).at[:, :K].set(kernel_points.T).reshape(3 * _L)

    sc = _make_sc_kernel(N, E, K, D_OUT)
    partials = sc(y.reshape(K * N, D_OUT), src, dst,
                  pos[:, 0], pos[:, 1], pos[:, 2], kp)
    return partials[0] + partials[1]


# speedup vs baseline: 1.0500x; 1.0500x over previous
import functools

import jax
import jax.numpy as jnp
from jax import lax
from jax.experimental import pallas as pl
from jax.experimental.pallas import tpu as pltpu
from jax.experimental.pallas import tpu_sc as plsc

_SIGMA = 0.5
_L = 16


def _y_matmul_body(x_ref, w_ref, o_ref):
    o_ref[0] = jnp.dot(x_ref[...], w_ref[0], preferred_element_type=jnp.float32)


def _rsqrt(d2):
    i = plsc.bitcast(d2, jnp.int32)
    i = jnp.int32(0x5F3759DF) - (i >> 1)
    y = plsc.bitcast(i, jnp.float32)
    for _ in range(2):
        y = y * (1.5 - 0.5 * d2 * y * y)
    return y


@functools.lru_cache(maxsize=None)
def _make_sc_kernel(N, E, K, D):
    NC, NS = 2, 16
    NW = NC * NS
    EPW = E // NW
    GPW = EPW // _L
    RPT = N // NS
    ZR = 25
    R8 = D // _L
    EB = 2000
    GPB = EB // _L
    CAP = 256
    NSLOT = 4

    mesh = plsc.VectorSubcoreMesh(core_axis_name="c", subcore_axis_name="s")

    @functools.partial(
        pl.kernel,
        out_type=jax.ShapeDtypeStruct((NC, N, D), jnp.float32),
        mesh=mesh,
        compiler_params=pltpu.CompilerParams(
            use_tc_tiling_on_sc=False, needs_layout_passes=False),
        scratch_types=[
            pltpu.VMEM((3 * _L,), jnp.float32),
            pltpu.VMEM((EB,), jnp.int32),
            pltpu.VMEM((EB,), jnp.int32),
            pltpu.VMEM((2, 6, _L), jnp.float32),
            pltpu.VMEM((2, CAP), jnp.int32),
            pltpu.VMEM((2, CAP), jnp.float32),
            pltpu.VMEM((2, CAP), jnp.int32),
            pltpu.VMEM((NSLOT, _L, D), jnp.float32),
            pltpu.VMEM((ZR, D), jnp.float32),
            pltpu.VMEM_SHARED((N, D), jnp.float32),
            pltpu.SemaphoreType.DMA,
            pltpu.SemaphoreType.DMA,
            pltpu.SemaphoreType.DMA,
            pltpu.SemaphoreType.DMA,
            pltpu.SemaphoreType.DMA,
            pltpu.SemaphoreType.DMA,
            pltpu.SemaphoreType.DMA,
            pltpu.SemaphoreType.DMA,
            pltpu.SemaphoreType.DMA,
        ],
    )
    def sc(y_hbm, src_hbm, dst_hbm, px_hbm, py_hbm, pz_hbm, kp_hbm, out_hbm,
           kpv, srcv, dstv, posb, idxb, inflb, dstb, rowb, zerov,
           outacc, psem, g0, g1, g2, g3, s0, s1, s2, s3):
        gsems = [g0, g1, g2, g3]
        ssems = [s0, s1, s2, s3]
        cid = lax.axis_index("c")
        sid = lax.axis_index("s")
        wid = cid * NS + sid
        ebase = wid * EPW

        pltpu.sync_copy(kp_hbm, kpv)

        def zb(i, _):
            for r in range(R8):
                zerov[i, pl.ds(r * _L, _L)] = jnp.zeros((_L,), jnp.float32)
            return 0
        lax.fori_loop(0, ZR, zb, 0)
        base = sid * RPT

        def zc(i, _):
            pltpu.sync_copy(zerov, outacc.at[pl.ds(base + i * ZR, ZR)])
            return 0
        lax.fori_loop(0, RPT // ZR, zc, 0)
        plsc.subcore_barrier()

        kxv = kpv[pl.ds(0, _L)]
        kyv = kpv[pl.ds(_L, _L)]
        kzv = kpv[pl.ds(2 * _L, _L)]
        kps = [(kxv[k], kyv[k], kzv[k]) for k in range(K)]

        def issue_pos(gl, b):
            sv = srcv[pl.ds(gl * _L, _L)]
            dv = dstv[pl.ds(gl * _L, _L)]
            pltpu.async_copy(px_hbm.at[sv], posb.at[b].at[0], psem)
            pltpu.async_copy(py_hbm.at[sv], posb.at[b].at[1], psem)
            pltpu.async_copy(pz_hbm.at[sv], posb.at[b].at[2], psem)
            pltpu.async_copy(px_hbm.at[dv], posb.at[b].at[3], psem)
            pltpu.async_copy(py_hbm.at[dv], posb.at[b].at[4], psem)
            pltpu.async_copy(pz_hbm.at[dv], posb.at[b].at[5], psem)

        def fire(pbuf, i, tb):
            iv = idxb[pbuf, pl.ds(i * _L, _L)]
            slot = (tb + i) % NSLOT
            for s in range(NSLOT):
                @pl.when(slot == s)
                def _():
                    @pl.when(tb + i >= NSLOT)
                    def _():
                        pltpu.make_async_copy(
                            y_hbm.at[pl.ds(0, _L)], rowb.at[s], ssems[s]).wait()
                    pltpu.async_copy(y_hbm.at[iv], rowb.at[s], gsems[s])

        def process(i, tb, inflv, dvec):
            slot = (tb + i) % NSLOT
            for s in range(NSLOT):
                @pl.when(slot == s)
                def _():
                    pltpu.make_async_copy(
                        y_hbm.at[pl.ds(0, _L)], rowb.at[s], gsems[s]).wait()
                    for j in range(_L):
                        f = inflv[j]
                        for r in range(R8):
                            rowb[s, j, pl.ds(r * _L, _L)] = (
                                rowb[s, j, pl.ds(r * _L, _L)] * f)
                    pltpu.async_copy(rowb.at[s], outacc.at[dvec],
                                     ssems[s], add=True)

        def run_batches(pbuf, pnb, tb):
            def bloop(i, _):
                @pl.when(i + (NSLOT - 1) < pnb)
                def _():
                    fire(pbuf, i + (NSLOT - 1), tb)
                inflv = inflb[pbuf, pl.ds(i * _L, _L)]
                dvec = dstb[pbuf, pl.ds(i * _L, _L)]
                process(i, tb, inflv, dvec)
                return 0
            lax.fori_loop(0, pnb, bloop, 0)

        def body(g, carry):
            rem, pnb, tb = carry
            gl = g % GPB
            b = g % 2
            pb = 1 - b

            run_batches(pb, pnb, tb)

            @pl.when(gl == 0)
            def _():
                blk = g // GPB
                pltpu.sync_copy(src_hbm.at[pl.ds(ebase + blk * EB, EB)], srcv)
                pltpu.sync_copy(dst_hbm.at[pl.ds(ebase + blk * EB, EB)], dstv)
                issue_pos(0, b)

            sv = srcv[pl.ds(gl * _L, _L)]
            dv = dstv[pl.ds(gl * _L, _L)]

            for c in range(6):
                pltpu.make_async_copy(
                    px_hbm.at[pl.ds(0, _L)], posb.at[b].at[c], psem).wait()
            relx = posb[b, 0, :] - posb[b, 3, :]
            rely = posb[b, 1, :] - posb[b, 4, :]
            relz = posb[b, 2, :] - posb[b, 5, :]

            @pl.when(jnp.logical_and(g + 1 < GPW, gl + 1 < GPB))
            def _():
                issue_pos(gl + 1, 1 - b)

            cnt = rem
            off = pnb * _L
            idxb[b, pl.ds(0, _L)] = idxb[pb, pl.ds(off, _L)]
            inflb[b, pl.ds(0, _L)] = inflb[pb, pl.ds(off, _L)]
            dstb[b, pl.ds(0, _L)] = dstb[pb, pl.ds(off, _L)]

            for k in range(K):
                kx, ky, kz = kps[k]
                dx = relx - kx
                dy = rely - ky
                dz = relz - kz
                d2 = dx * dx + dy * dy + dz * dz + 1e-12
                dist = d2 * _rsqrt(d2)
                infl = jnp.maximum(0.0, 1.0 - dist * (1.0 / _SIGMA))
                m = infl > 0.0
                plsc.store_compressed(
                    idxb.at[b].at[pl.ds(cnt, _L)], sv + k * N, mask=m)
                plsc.store_compressed(
                    inflb.at[b].at[pl.ds(cnt, _L)], infl, mask=m)
                plsc.store_compressed(
                    dstb.at[b].at[pl.ds(cnt, _L)], dv, mask=m)
                cnt = cnt + plsc.all_reduce_population_count(m)[0]

            nbat = cnt // _L
            tbn = tb + pnb

            def prime(i, _):
                fire(b, i, tbn)
                return 0
            lax.fori_loop(0, jnp.minimum(nbat, NSLOT - 1), prime, 0)

            return cnt - nbat * _L, nbat, tbn

        rem, pnb, tb = lax.fori_loop(
            0, GPW, body, (jnp.int32(0), jnp.int32(0), jnp.int32(0)))

        lb = 1 - (GPW % 2)
        run_batches(lb, pnb, tb)

        @pl.when(rem > 0)
        def _():
            off = pnb * _L
            lane = lax.iota(jnp.int32, _L)
            m = lane < rem
            iv = jnp.where(m, idxb[lb, pl.ds(off, _L)], 0)
            fv = jnp.where(m, inflb[lb, pl.ds(off, _L)], 0.0)
            dvv = jnp.where(m, dstb[lb, pl.ds(off, _L)], 0)
            idxb[lb, pl.ds(off, _L)] = iv
            inflb[lb, pl.ds(off, _L)] = fv
            dstb[lb, pl.ds(off, _L)] = dvv
            fire(lb, pnb, tb)
            process(pnb, tb, fv, dvv)

        tbf = tb + pnb + jnp.where(rem > 0, 1, 0)
        for s in range(NSLOT):
            @pl.when(tbf > s)
            def _():
                pltpu.make_async_copy(
                    y_hbm.at[pl.ds(0, _L)], rowb.at[s], ssems[s]).wait()
        plsc.subcore_barrier()

        pltpu.sync_copy(outacc.at[pl.ds(base, RPT)],
                        out_hbm.at[cid].at[pl.ds(base, RPT)])

    return sc


def kernel(x, pos, edge_index, kernel_points, weights):
    N, D_IN = x.shape
    K = kernel_points.shape[0]
    D_OUT = weights.shape[2]
    E = edge_index.shape[1]

    y = pl.pallas_call(
        _y_matmul_body,
        grid=(K,),
        in_specs=[
            pl.BlockSpec((N, D_IN), lambda k: (0, 0)),
            pl.BlockSpec((1, D_IN, D_OUT), lambda k: (k, 0, 0)),
        ],
        out_specs=pl.BlockSpec((1, N, D_OUT), lambda k: (k, 0, 0)),
        out_shape=jax.ShapeDtypeStruct((K, N, D_OUT), jnp.float32),
    )(x, weights)

    src = edge_index[0]
    dst = edge_index[1]
    kp = jnp.zeros((3, _L), jnp.float32).at[:, :K].set(kernel_points.T).reshape(3 * _L)

    sc = _make_sc_kernel(N, E, K, D_OUT)
    partials = sc(y.reshape(K * N, D_OUT), src, dst,
                  pos[:, 0], pos[:, 1], pos[:, 2], kp)
    return partials[0] + partials[1]
